# Initial kernel scaffold; baseline (speedup 1.0000x reference)
import jax
import jax.numpy as jnp
from jax.experimental import pallas as pl


def kernel(inputs, token_table, pos_table):
    raise NotImplementedError("write your pallas kernel here")


# speedup vs baseline: 3.4063x; 3.4063x over previous
import functools

import jax
import jax.numpy as jnp
from jax import lax
from jax.experimental import pallas as pl
from jax.experimental.pallas import tpu as pltpu
from jax.experimental.pallas import tpu_sc as plsc

VOCAB = 100000
MAX_LEN = 200
EMBED_DIM = 64
BATCH = 4096

NC, NS, L = 2, 16, 16
NW = NC * NS
TOTAL_ROWS = BATCH * MAX_LEN
ROWS_PER_W = TOTAL_ROWS // NW
GATHER = 100
SEQS_PER_W = BATCH // NW
IDX_ROWS_PER_W = ROWS_PER_W // GATHER
NBUF = 4
LEAD = 2


def _sc_embed(idx_hbm, table_hbm, pos_hbm):
    mesh = plsc.VectorSubcoreMesh(
        core_axis_name="c", subcore_axis_name="s", num_cores=NC, num_subcores=NS
    )

    @functools.partial(
        pl.kernel,
        mesh=mesh,
        out_type=jax.ShapeDtypeStruct((BATCH, MAX_LEN, EMBED_DIM), jnp.float32),
        compiler_params=pltpu.CompilerParams(use_tc_tiling_on_sc=False),
        scratch_types=[
            pltpu.VMEM((IDX_ROWS_PER_W, GATHER), jnp.int32),
            pltpu.VMEM((MAX_LEN, EMBED_DIM), jnp.float32),
            pltpu.VMEM((NBUF, MAX_LEN, EMBED_DIM), jnp.float32),
            [pltpu.SemaphoreType.DMA] * NBUF,
            [pltpu.SemaphoreType.DMA] * NBUF,
        ],
    )
    def k(idx_ref, table_ref, pos_ref, out_ref, idx_v, pos_v, rows_v, gsems, osems):
        wid = lax.axis_index("s") * NC + lax.axis_index("c")
        pltpu.sync_copy(idx_ref.at[pl.ds(wid * IDX_ROWS_PER_W, IDX_ROWS_PER_W)], idx_v)
        pltpu.sync_copy(pos_ref, pos_v)
        base_seq = wid * SEQS_PER_W

        def gather_descs(s, b):
            buf = rows_v.at[b]
            return (
                pltpu.make_async_copy(
                    table_ref.at[idx_v.at[2 * s]], buf.at[pl.ds(0, GATHER)], gsems[b]
                ),
                pltpu.make_async_copy(
                    table_ref.at[idx_v.at[2 * s + 1]],
                    buf.at[pl.ds(GATHER, GATHER)],
                    gsems[b],
                ),
            )

        def scatter_desc(s, b):
            return pltpu.make_async_copy(
                rows_v.at[b], out_ref.at[base_seq + s], osems[b]
            )

        def issue_gather(s, b):
            for d in gather_descs(s, b):
                d.start()

        for b in range(LEAD):
            issue_gather(b, b)

        def outer_body(t, _):
            for kk in range(NBUF):
                s = t * NBUF + kk
                for d in gather_descs(s, kk):
                    d.wait()

                def row_body(r, _):
                    for j in range(EMBED_DIM // L):
                        sl = pl.ds(j * L, L)
                        rows_v[kk, r, sl] = rows_v[kk, r, sl] + pos_v[r, sl]
                    return 0

                lax.fori_loop(0, MAX_LEN, row_body, 0, unroll=2)

                scatter_desc(s, kk).start()

                bn = (kk + LEAD) % NBUF
                if kk < NBUF - LEAD:
                    @pl.when(t >= 1)
                    def _():
                        scatter_desc(s + LEAD - NBUF, bn).wait()

                    issue_gather(s + LEAD, bn)
                else:
                    @pl.when(t <= SEQS_PER_W // NBUF - 2)
                    def _():
                        scatter_desc(s + LEAD - NBUF, bn).wait()
                        issue_gather(s + LEAD, bn)
            return 0

        lax.fori_loop(0, SEQS_PER_W // NBUF, outer_body, 0)

        for b in range(NBUF):
            scatter_desc(SEQS_PER_W - NBUF + b, b).wait()

    return k(idx_hbm, table_hbm, pos_hbm)


def kernel(inputs, token_table, pos_table):
    idx = inputs.reshape(-1).astype(jnp.int32).reshape(TOTAL_ROWS // GATHER, GATHER)
    return _sc_embed(idx, token_table, pos_table)


# speedup vs baseline: 3.6421x; 1.0692x over previous
import functools

import jax
import jax.numpy as jnp
from jax import lax
from jax.experimental import pallas as pl
from jax.experimental.pallas import tpu as pltpu
from jax.experimental.pallas import tpu_sc as plsc

VOCAB = 100000
MAX_LEN = 200
EMBED_DIM = 64
BATCH = 4096

NC, NS, L = 2, 16, 16
NW = NC * NS
TOTAL_ROWS = BATCH * MAX_LEN
ROWS_PER_W = TOTAL_ROWS // NW
GATHER = 100
SEQS_PER_W = BATCH // NW
IDX_ROWS_PER_W = ROWS_PER_W // GATHER
NBUF = 8
LEAD = 4
NCHUNKS_W = IDX_ROWS_PER_W


def _sc_embed(idx_hbm, table_hbm, pos_hbm):
    mesh = plsc.VectorSubcoreMesh(
        core_axis_name="c", subcore_axis_name="s", num_cores=NC, num_subcores=NS
    )

    @functools.partial(
        pl.kernel,
        mesh=mesh,
        out_type=jax.ShapeDtypeStruct((BATCH, MAX_LEN, EMBED_DIM), jnp.float32),
        compiler_params=pltpu.CompilerParams(use_tc_tiling_on_sc=False),
        scratch_types=[
            pltpu.VMEM((IDX_ROWS_PER_W, GATHER), jnp.int32),
            pltpu.VMEM((MAX_LEN, EMBED_DIM), jnp.float32),
            pltpu.VMEM((NBUF, GATHER, EMBED_DIM), jnp.float32),
            [pltpu.SemaphoreType.DMA] * NBUF,
            [pltpu.SemaphoreType.DMA] * NBUF,
        ],
    )
    def k(idx_ref, table_ref, pos_ref, out_ref, idx_v, pos_v, rows_v, gsems, osems):
        wid = lax.axis_index("s") * NC + lax.axis_index("c")
        pltpu.sync_copy(idx_ref.at[pl.ds(wid * IDX_ROWS_PER_W, IDX_ROWS_PER_W)], idx_v)
        pltpu.sync_copy(pos_ref, pos_v)
        base_seq = wid * SEQS_PER_W

        def gather_desc(c, b):
            return pltpu.make_async_copy(
                table_ref.at[idx_v.at[c]], rows_v.at[b], gsems[b]
            )

        def scatter_desc(c, b):
            return pltpu.make_async_copy(
                rows_v.at[b],
                out_ref.at[base_seq + c // 2].at[pl.ds((c % 2) * GATHER, GATHER)],
                osems[b],
            )

        for b in range(LEAD):
            gather_desc(b, b).start()

        def outer_body(t, _):
            for kk in range(NBUF):
                c = t * NBUF + kk
                h = kk % 2
                gather_desc(c, kk).wait()

                def row_body(r, _):
                    for j in range(EMBED_DIM // L):
                        sl = pl.ds(j * L, L)
                        rows_v[kk, r, sl] = rows_v[kk, r, sl] + pos_v[h * GATHER + r, sl]
                    return 0

                lax.fori_loop(0, GATHER, row_body, 0, unroll=2)

                scatter_desc(c, kk).start()

                bn = (kk + LEAD) % NBUF
                if kk < NBUF - LEAD:
                    @pl.when(t >= 1)
                    def _():
                        scatter_desc(c + LEAD - NBUF, bn).wait()

                    gather_desc(c + LEAD, bn).start()
                else:
                    @pl.when(t <= NCHUNKS_W // NBUF - 2)
                    def _():
                        scatter_desc(c + LEAD - NBUF, bn).wait()
                        gather_desc(c + LEAD, bn).start()
            return 0

        lax.fori_loop(0, NCHUNKS_W // NBUF, outer_body, 0)

        for i in range(NBUF):
            c = NCHUNKS_W - NBUF + i
            scatter_desc(c, c % NBUF).wait()

    return k(idx_hbm, table_hbm, pos_hbm)


def kernel(inputs, token_table, pos_table):
    idx = inputs.reshape(-1).astype(jnp.int32).reshape(TOTAL_ROWS // GATHER, GATHER)
    return _sc_embed(idx, token_table, pos_table)


# speedup vs baseline: 3.6503x; 1.0023x over previous
import functools

import jax
import jax.numpy as jnp
from jax import lax
from jax.experimental import pallas as pl
from jax.experimental.pallas import tpu as pltpu
from jax.experimental.pallas import tpu_sc as plsc

VOCAB = 100000
MAX_LEN = 200
EMBED_DIM = 64
BATCH = 4096

NC, NS, L = 2, 16, 16
NW = NC * NS
TOTAL_ROWS = BATCH * MAX_LEN
ROWS_PER_W = TOTAL_ROWS // NW
GATHER = 100
SEQS_PER_W = BATCH // NW
IDX_ROWS_PER_W = ROWS_PER_W // GATHER
NBUF = 8
LEAD = 6
NCHUNKS_W = IDX_ROWS_PER_W


def _sc_embed(idx_hbm, table_hbm, pos_hbm):
    mesh = plsc.VectorSubcoreMesh(
        core_axis_name="c", subcore_axis_name="s", num_cores=NC, num_subcores=NS
    )

    @functools.partial(
        pl.kernel,
        mesh=mesh,
        out_type=jax.ShapeDtypeStruct((BATCH, MAX_LEN, EMBED_DIM), jnp.float32),
        compiler_params=pltpu.CompilerParams(use_tc_tiling_on_sc=False),
        scratch_types=[
            pltpu.VMEM((IDX_ROWS_PER_W, GATHER), jnp.int32),
            pltpu.VMEM((MAX_LEN, EMBED_DIM), jnp.float32),
            pltpu.VMEM((NBUF, GATHER, EMBED_DIM), jnp.float32),
            [pltpu.SemaphoreType.DMA] * NBUF,
            [pltpu.SemaphoreType.DMA] * NBUF,
        ],
    )
    def k(idx_ref, table_ref, pos_ref, out_ref, idx_v, pos_v, rows_v, gsems, osems):
        wid = lax.axis_index("s") * NC + lax.axis_index("c")
        pltpu.sync_copy(idx_ref.at[pl.ds(wid * IDX_ROWS_PER_W, IDX_ROWS_PER_W)], idx_v)
        pltpu.sync_copy(pos_ref, pos_v)
        base_seq = wid * SEQS_PER_W

        def gather_desc(c, b):
            return pltpu.make_async_copy(
                table_ref.at[idx_v.at[c]], rows_v.at[b], gsems[b]
            )

        def scatter_desc(c, b):
            return pltpu.make_async_copy(
                rows_v.at[b],
                out_ref.at[base_seq + c // 2].at[pl.ds((c % 2) * GATHER, GATHER)],
                osems[b],
            )

        for b in range(LEAD):
            gather_desc(b, b).start()

        def outer_body(t, _):
            for kk in range(NBUF):
                c = t * NBUF + kk
                h = kk % 2
                gather_desc(c, kk).wait()

                def row_body(r, _):
                    for j in range(EMBED_DIM // L):
                        sl = pl.ds(j * L, L)
                        rows_v[kk, r, sl] = rows_v[kk, r, sl] + pos_v[h * GATHER + r, sl]
                    return 0

                lax.fori_loop(0, GATHER, row_body, 0, unroll=4)

                scatter_desc(c, kk).start()

                bn = (kk + LEAD) % NBUF
                if kk < NBUF - LEAD:
                    @pl.when(t >= 1)
                    def _():
                        scatter_desc(c + LEAD - NBUF, bn).wait()

                    gather_desc(c + LEAD, bn).start()
                else:
                    @pl.when(t <= NCHUNKS_W // NBUF - 2)
                    def _():
                        scatter_desc(c + LEAD - NBUF, bn).wait()
                        gather_desc(c + LEAD, bn).start()
            return 0

        lax.fori_loop(0, NCHUNKS_W // NBUF, outer_body, 0)

        for i in range(NBUF):
            c = NCHUNKS_W - NBUF + i
            scatter_desc(c, c % NBUF).wait()

    return k(idx_hbm, table_hbm, pos_hbm)


def kernel(inputs, token_table, pos_table):
    idx = inputs.reshape(-1).astype(jnp.int32).reshape(TOTAL_ROWS // GATHER, GATHER)
    return _sc_embed(idx, token_table, pos_table)


# speedup vs baseline: 3.8853x; 1.0644x over previous
import functools

import jax
import jax.numpy as jnp
from jax import lax
from jax.experimental import pallas as pl
from jax.experimental.pallas import tpu as pltpu
from jax.experimental.pallas import tpu_sc as plsc

VOCAB = 100000
MAX_LEN = 200
EMBED_DIM = 64
BATCH = 4096

NC, NS, L = 2, 16, 16
NW = NC * NS
TOTAL_ROWS = BATCH * MAX_LEN
ROWS_PER_W = TOTAL_ROWS // NW
GATHER = 100
SEQS_PER_W = BATCH // NW
IDX_ROWS_PER_W = ROWS_PER_W // GATHER
NBUF = 8
LEAD = 6
NPBUF = 4
NCHUNKS_W = IDX_ROWS_PER_W
DB = EMBED_DIM // 8
NJ = BATCH // 128


def _sc_embed(idx_hbm, table_hbm, pos_hbm):
    mesh = plsc.VectorSubcoreMesh(
        core_axis_name="c", subcore_axis_name="s", num_cores=NC, num_subcores=NS
    )

    @functools.partial(
        pl.kernel,
        mesh=mesh,
        out_type=jax.ShapeDtypeStruct((BATCH, MAX_LEN * EMBED_DIM), jnp.float32),
        compiler_params=pltpu.CompilerParams(use_tc_tiling_on_sc=False),
        scratch_types=[
            pltpu.VMEM((IDX_ROWS_PER_W, GATHER), jnp.int32),
            pltpu.VMEM((MAX_LEN, EMBED_DIM), jnp.float32),
            pltpu.VMEM((NBUF, GATHER, EMBED_DIM), jnp.float32),
            pltpu.VMEM((NPBUF, GATHER * EMBED_DIM), jnp.float32),
            [pltpu.SemaphoreType.DMA] * NBUF,
            [pltpu.SemaphoreType.DMA] * NPBUF,
        ],
    )
    def k(idx_ref, table_ref, pos_ref, out_ref,
          idx_v, pos_v, gbuf, pbuf, gsems, osems):
        wid = lax.axis_index("s") * NC + lax.axis_index("c")
        pltpu.sync_copy(idx_ref.at[pl.ds(wid * IDX_ROWS_PER_W, IDX_ROWS_PER_W)], idx_v)
        pltpu.sync_copy(pos_ref, pos_v)
        base_b = wid * SEQS_PER_W

        def gather_desc(c, b):
            return pltpu.make_async_copy(
                table_ref.at[idx_v.at[c]], gbuf.at[b], gsems[b]
            )

        def scatter_desc(c, b):
            return pltpu.make_async_copy(
                pbuf.at[b],
                out_ref.at[
                    base_b + c // 2,
                    pl.ds((c % 2) * GATHER * EMBED_DIM, GATHER * EMBED_DIM),
                ],
                osems[b],
            )

        for b in range(LEAD):
            gather_desc(b, b).start()

        def outer_body(t, _):
            for kk in range(NBUF):
                c = t * NBUF + kk
                h = kk % 2
                pb = kk % NPBUF
                if kk < NPBUF:
                    @pl.when(t >= 1)
                    def _():
                        scatter_desc(c - NPBUF, pb).wait()
                else:
                    scatter_desc(c - NPBUF, pb).wait()
                gather_desc(c, kk).wait()

                def m_body(m, _):
                    p0 = h * GATHER + 2 * m
                    for j in range(EMBED_DIM // L):
                        sl = pl.ds(j * L, L)
                        pbuf[pb, pl.ds(m * 2 * EMBED_DIM + j * L, L)] = (
                            gbuf[kk, 2 * m, sl] + pos_v[p0, sl]
                        )
                        pbuf[pb, pl.ds(m * 2 * EMBED_DIM + EMBED_DIM + j * L, L)] = (
                            gbuf[kk, 2 * m + 1, sl] + pos_v[p0 + 1, sl]
                        )
                    return 0

                lax.fori_loop(0, GATHER // 2, m_body, 0, unroll=4)

                scatter_desc(c, pb).start()

                bn = (kk + LEAD) % NBUF
                if kk < NBUF - LEAD:
                    gather_desc(c + LEAD, bn).start()
                else:
                    @pl.when(t <= NCHUNKS_W // NBUF - 2)
                    def _():
                        gather_desc(c + LEAD, bn).start()
            return 0

        lax.fori_loop(0, NCHUNKS_W // NBUF, outer_body, 0)

        for i in range(NPBUF):
            c = NCHUNKS_W - NPBUF + i
            scatter_desc(c, c % NPBUF).wait()

    return k(idx_hbm, table_hbm, pos_hbm)


def _tc_finalize(x2d):
    def body(x_ref, o_ref):
        for h in range(2):
            for j in range(NJ):
                blk = x_ref[
                    pl.ds(j * 128, 128), pl.ds(h * EMBED_DIM, EMBED_DIM)
                ]
                o_ref[h, :, j] = jnp.transpose(blk, (1, 0)).reshape(DB, 8, 128)

    return pl.pallas_call(
        body,
        grid=(MAX_LEN // 2,),
        in_specs=[pl.BlockSpec((BATCH, 128), lambda i: (0, i))],
        out_specs=pl.BlockSpec(
            (2, DB, NJ, 8, 128), lambda i: (i, 0, 0, 0, 0)
        ),
        out_shape=jax.ShapeDtypeStruct((MAX_LEN, DB, NJ, 8, 128), jnp.float32),
    )(x2d)


def kernel(inputs, token_table, pos_table):
    idx = inputs.reshape(-1).astype(jnp.int32).reshape(TOTAL_ROWS // GATHER, GATHER)
    out2 = _sc_embed(idx, token_table, pos_table)
    out5 = _tc_finalize(out2)
    return jnp.transpose(out5, (2, 4, 0, 1, 3)).reshape(BATCH, MAX_LEN, EMBED_DIM)


# speedup vs baseline: 5.2194x; 1.3434x over previous
import functools

import jax
import jax.numpy as jnp
from jax import lax
from jax.experimental import pallas as pl
from jax.experimental.pallas import tpu as pltpu
from jax.experimental.pallas import tpu_sc as plsc

VOCAB = 100000
MAX_LEN = 200
EMBED_DIM = 64
BATCH = 4096

NC, NS, L = 2, 16, 16
NW = NC * NS
TOTAL_ROWS = BATCH * MAX_LEN
ROWS_PER_W = TOTAL_ROWS // NW
GATHER = 100
SEQS_PER_W = BATCH // NW
IDX_ROWS_PER_W = ROWS_PER_W // GATHER
NBUF = 8
LEAD = 6
NPBUF = 4
NCHUNKS_W = IDX_ROWS_PER_W
DB = EMBED_DIM // 8
NJ = BATCH // 128


def _sc_embed(idx_hbm, table_hbm, pos_hbm):
    mesh = plsc.VectorSubcoreMesh(
        core_axis_name="c", subcore_axis_name="s", num_cores=NC, num_subcores=NS
    )

    @functools.partial(
        pl.kernel,
        mesh=mesh,
        out_type=jax.ShapeDtypeStruct((MAX_LEN // 2, BATCH, 2 * EMBED_DIM), jnp.float32),
        compiler_params=pltpu.CompilerParams(use_tc_tiling_on_sc=False),
        scratch_types=[
            pltpu.VMEM((IDX_ROWS_PER_W, GATHER), jnp.int32),
            pltpu.VMEM((MAX_LEN, EMBED_DIM), jnp.float32),
            pltpu.VMEM((NBUF, GATHER, EMBED_DIM), jnp.float32),
            pltpu.VMEM((NPBUF, GATHER // 2, 2 * EMBED_DIM), jnp.float32),
            [pltpu.SemaphoreType.DMA] * NBUF,
            [pltpu.SemaphoreType.DMA] * NPBUF,
        ],
    )
    def k(idx_ref, table_ref, pos_ref, out_ref,
          idx_v, pos_v, gbuf, pbuf, gsems, osems):
        wid = lax.axis_index("s") * NC + lax.axis_index("c")
        pltpu.sync_copy(idx_ref.at[pl.ds(wid * IDX_ROWS_PER_W, IDX_ROWS_PER_W)], idx_v)
        pltpu.sync_copy(pos_ref, pos_v)
        base_b = wid * SEQS_PER_W

        def gather_desc(c, b):
            return pltpu.make_async_copy(
                table_ref.at[idx_v.at[c]], gbuf.at[b], gsems[b]
            )

        def scatter_desc(c, b):
            return pltpu.make_async_copy(
                pbuf.at[b],
                out_ref.at[
                    pl.ds((c % 2) * (GATHER // 2), GATHER // 2), base_b + c // 2
                ],
                osems[b],
            )

        for b in range(LEAD):
            gather_desc(b, b).start()

        def outer_body(t, _):
            for kk in range(NBUF):
                c = t * NBUF + kk
                h = kk % 2
                pb = kk % NPBUF
                if kk < NPBUF:
                    @pl.when(t >= 1)
                    def _():
                        scatter_desc(c - NPBUF, pb).wait()
                else:
                    scatter_desc(c - NPBUF, pb).wait()
                gather_desc(c, kk).wait()

                def m_body(m, _):
                    p0 = h * GATHER + 2 * m
                    for j in range(EMBED_DIM // L):
                        sl = pl.ds(j * L, L)
                        pbuf[pb, m, sl] = gbuf[kk, 2 * m, sl] + pos_v[p0, sl]
                        pbuf[pb, m, pl.ds(EMBED_DIM + j * L, L)] = (
                            gbuf[kk, 2 * m + 1, sl] + pos_v[p0 + 1, sl]
                        )
                    return 0

                lax.fori_loop(0, GATHER // 2, m_body, 0, unroll=4)

                scatter_desc(c, pb).start()

                bn = (kk + LEAD) % NBUF
                if kk < NBUF - LEAD:
                    gather_desc(c + LEAD, bn).start()
                else:
                    @pl.when(t <= NCHUNKS_W // NBUF - 2)
                    def _():
                        gather_desc(c + LEAD, bn).start()
            return 0

        lax.fori_loop(0, NCHUNKS_W // NBUF, outer_body, 0)

        for i in range(NPBUF):
            c = NCHUNKS_W - NPBUF + i
            scatter_desc(c, c % NPBUF).wait()

    return k(idx_hbm, table_hbm, pos_hbm)


def _tc_finalize(x2d):
    def body(x_ref, o_ref):
        for h in range(2):
            for j in range(NJ):
                blk = x_ref[
                    0, pl.ds(j * 128, 128), pl.ds(h * EMBED_DIM, EMBED_DIM)
                ]
                o_ref[h, :, j] = jnp.transpose(blk, (1, 0)).reshape(DB, 8, 128)

    return pl.pallas_call(
        body,
        grid=(MAX_LEN // 2,),
        in_specs=[pl.BlockSpec((1, BATCH, 128), lambda i: (i, 0, 0))],
        out_specs=pl.BlockSpec(
            (2, DB, NJ, 8, 128), lambda i: (i, 0, 0, 0, 0)
        ),
        out_shape=jax.ShapeDtypeStruct((MAX_LEN, DB, NJ, 8, 128), jnp.float32),
    )(x2d)


def kernel(inputs, token_table, pos_table):
    idx = inputs.reshape(-1).astype(jnp.int32).reshape(TOTAL_ROWS // GATHER, GATHER)
    out3 = _sc_embed(idx, token_table, pos_table)
    out5 = _tc_finalize(out3)
    return jnp.transpose(out5, (2, 4, 0, 1, 3)).reshape(BATCH, MAX_LEN, EMBED_DIM)
